# Initial kernel scaffold; baseline (speedup 1.0000x reference)
import jax
import jax.numpy as jnp
from jax.experimental import pallas as pl


def kernel(x, W, b, proj):
    raise NotImplementedError("write your pallas kernel here")


# speedup vs baseline: 1.7448x; 1.7448x over previous
import jax
import jax.numpy as jnp
import numpy as np
from jax.experimental import pallas as pl

_T, _H = 8, 8
_D = 1024
_N = 4096

_PMAT = np.zeros((_T * _H, 128), np.float32)
for _t in range(_T):
    for _h in range(_H):
        _PMAT[_t * _H + _h, _t] = float(2 ** _h)


def _codes_body(rows_ref, projT_ref, pmat_ref, out_ref):
    dots = jnp.dot(rows_ref[...], projT_ref[...],
                   preferred_element_type=jnp.float32)
    bits = (dots > 0).astype(jnp.float32)
    out_ref[...] = jnp.dot(bits, pmat_ref[...],
                           preferred_element_type=jnp.float32).astype(jnp.int32)


def _compute_codes(rows, projT, pmat):
    n_rows = rows.shape[0]
    tile = 1024
    return pl.pallas_call(
        _codes_body,
        grid=(n_rows // tile,),
        in_specs=[
            pl.BlockSpec((tile, _D), lambda i: (i, 0)),
            pl.BlockSpec((_D, _T * _H), lambda i: (0, 0)),
            pl.BlockSpec((_T * _H, 128), lambda i: (0, 0)),
        ],
        out_specs=pl.BlockSpec((tile, 128), lambda i: (i, 0)),
        out_shape=jax.ShapeDtypeStruct((n_rows, 128), jnp.int32),
    )(rows, projT, pmat)


_TS, _TN = 512, 512


def _masked_linear_body(x_ref, w_ref, b_ref, hx_ref, hw_ref, out_ref):
    dense = jax.lax.dot_general(
        x_ref[...], w_ref[...],
        dimension_numbers=(((1,), (1,)), ((), ())),
        preferred_element_type=jnp.float32)
    dense = dense + b_ref[...]
    mask = hx_ref[:, 0:1] == hw_ref[0:1, :]
    for t in range(1, _T):
        mask = mask | (hx_ref[:, t:t + 1] == hw_ref[t:t + 1, :])
    out_ref[...] = jnp.where(mask, dense, 0.0)


def kernel(x, W, b, proj):
    B, S, D = x.shape
    BS = B * S
    xf = x.reshape(BS, D)
    projT = proj.reshape(_T * _H, D).T
    pmat = jnp.asarray(_PMAT)
    hx = _compute_codes(xf, projT, pmat)
    hw = _compute_codes(W, projT, pmat)
    hwT = hw[:, :_T].T
    b2 = b.reshape(1, _N)
    out = pl.pallas_call(
        _masked_linear_body,
        grid=(BS // _TS, _N // _TN),
        in_specs=[
            pl.BlockSpec((_TS, D), lambda i, j: (i, 0)),
            pl.BlockSpec((_TN, D), lambda i, j: (j, 0)),
            pl.BlockSpec((1, _TN), lambda i, j: (0, j)),
            pl.BlockSpec((_TS, 128), lambda i, j: (i, 0)),
            pl.BlockSpec((_T, _TN), lambda i, j: (0, j)),
        ],
        out_specs=pl.BlockSpec((_TS, _TN), lambda i, j: (i, j)),
        out_shape=jax.ShapeDtypeStruct((BS, _N), jnp.float32),
    )(xf, W, b2, hx, hwT)
    return out.reshape(B, S, _N)


# speedup vs baseline: 2.4173x; 1.3854x over previous
import jax
import jax.numpy as jnp
import numpy as np
from jax.experimental import pallas as pl
from jax.experimental.pallas import tpu as pltpu

_T, _H = 8, 8
_D = 1024
_N = 4096
_TS, _TN = 2048, 512

_PMAT = np.zeros((_T * _H, 128), np.float32)
for _t in range(_T):
    for _h in range(_H):
        _PMAT[_t * _H + _h, _t] = float(2 ** _h)
_PMAT_T8 = np.ascontiguousarray(_PMAT[:, :_T].T)


def _body(x_ref, w_ref, b_ref, projT_ref, projM_ref, pmat_ref, pmatT8_ref,
          out_ref, hx_s, hw_s):
    i = pl.program_id(0)
    j = pl.program_id(1)

    @pl.when(j == 0)
    def _():
        dots = jnp.dot(x_ref[...], projT_ref[...],
                       preferred_element_type=jnp.float32)
        bits = (dots > 0).astype(jnp.float32)
        hx_s[...] = jnp.dot(bits, pmat_ref[...],
                            preferred_element_type=jnp.float32).astype(jnp.int32)

    @pl.when(i == 0)
    def _():
        dw = jax.lax.dot_general(projM_ref[...], w_ref[...],
                                 dimension_numbers=(((1,), (1,)), ((), ())),
                                 preferred_element_type=jnp.float32)
        bw = (dw > 0).astype(jnp.float32)
        hw_s[:, pl.ds(j * _TN, _TN)] = jnp.dot(
            pmatT8_ref[...], bw,
            preferred_element_type=jnp.float32).astype(jnp.int32)

    dense = jax.lax.dot_general(x_ref[...], w_ref[...],
                                dimension_numbers=(((1,), (1,)), ((), ())),
                                preferred_element_type=jnp.float32)
    dense = dense + b_ref[...]
    hw_t = hw_s[:, pl.ds(j * _TN, _TN)]
    mask = hx_s[:, 0:1] == hw_t[0:1, :]
    for t in range(1, _T):
        mask = mask | (hx_s[:, t:t + 1] == hw_t[t:t + 1, :])
    out_ref[...] = jnp.where(mask, dense, 0.0)


def kernel(x, W, b, proj):
    B, S, D = x.shape
    BS = B * S
    xf = x.reshape(BS, D)
    projM = proj.reshape(_T * _H, D)
    projT = projM.T
    b2 = b.reshape(1, _N)
    out = pl.pallas_call(
        _body,
        grid=(BS // _TS, _N // _TN),
        in_specs=[
            pl.BlockSpec((_TS, D), lambda i, j: (i, 0)),
            pl.BlockSpec((_TN, D), lambda i, j: (j, 0)),
            pl.BlockSpec((1, _TN), lambda i, j: (0, j)),
            pl.BlockSpec((D, _T * _H), lambda i, j: (0, 0)),
            pl.BlockSpec((_T * _H, D), lambda i, j: (0, 0)),
            pl.BlockSpec((_T * _H, 128), lambda i, j: (0, 0)),
            pl.BlockSpec((_T, _T * _H), lambda i, j: (0, 0)),
        ],
        out_specs=pl.BlockSpec((_TS, _TN), lambda i, j: (i, j)),
        out_shape=jax.ShapeDtypeStruct((BS, _N), jnp.float32),
        scratch_shapes=[
            pltpu.VMEM((_TS, 128), jnp.int32),
            pltpu.VMEM((_T, _N), jnp.int32),
        ],
    )(xf, W, b2, jnp.asarray(projT), projM, jnp.asarray(_PMAT),
      jnp.asarray(_PMAT_T8))
    return out.reshape(B, S, _N)


# speedup vs baseline: 3.1191x; 1.2903x over previous
import jax
import jax.numpy as jnp
import numpy as np
from jax.experimental import pallas as pl
from jax.experimental.pallas import tpu as pltpu

_T, _H = 8, 8
_D = 1024
_N = 4096
_TS, _TN = 2048, 512

_PMAT = np.zeros((_T * _H, 128), np.float32)
for _t in range(_T):
    for _h in range(_H):
        _PMAT[_t * _H + _h, _t] = float(2 ** _h)
_PMAT_T8 = np.ascontiguousarray(_PMAT[:, :_T].T)


_C_ONES = np.int32(0x01010101)
_C_HIGH = np.int32(np.uint32(0x80808080).astype(np.int64) - (1 << 32))


def _pack4_cols(c):
    p0 = c[:, 0:1] | (c[:, 1:2] << 8) | (c[:, 2:3] << 16) | (c[:, 3:4] << 24)
    p1 = c[:, 4:5] | (c[:, 5:6] << 8) | (c[:, 6:7] << 16) | (c[:, 7:8] << 24)
    return p0, p1


def _body(x_ref, w_ref, b_ref, projT_ref, projM_ref, pmat_ref, pmatT8_ref,
          out_ref, hx_s, hw_s):
    i = pl.program_id(0)
    j = pl.program_id(1)

    @pl.when(j == 0)
    def _():
        dots = jnp.dot(x_ref[...], projT_ref[...],
                       preferred_element_type=jnp.float32)
        bits = (dots > 0).astype(jnp.float32)
        codes = jnp.dot(bits, pmat_ref[...],
                        preferred_element_type=jnp.float32).astype(jnp.int32)
        p0, p1 = _pack4_cols(codes)
        hx_s[:, 0:1] = p0
        hx_s[:, 1:2] = p1

    @pl.when(i == 0)
    def _():
        dw = jax.lax.dot_general(projM_ref[...], w_ref[...],
                                 dimension_numbers=(((1,), (1,)), ((), ())),
                                 preferred_element_type=jnp.float32)
        bw = (dw > 0).astype(jnp.float32)
        cw = jnp.dot(pmatT8_ref[...], bw,
                     preferred_element_type=jnp.float32).astype(jnp.int32)
        q0 = (cw[0:1, :] | (cw[1:2, :] << 8) | (cw[2:3, :] << 16)
              | (cw[3:4, :] << 24))
        q1 = (cw[4:5, :] | (cw[5:6, :] << 8) | (cw[6:7, :] << 16)
              | (cw[7:8, :] << 24))
        hw_s[0:1, pl.ds(j * _TN, _TN)] = q0
        hw_s[1:2, pl.ds(j * _TN, _TN)] = q1

    dense = jax.lax.dot_general(x_ref[...], w_ref[...],
                                dimension_numbers=(((1,), (1,)), ((), ())),
                                preferred_element_type=jnp.float32)
    v0 = hx_s[:, 0:1] ^ hw_s[0:1, pl.ds(j * _TN, _TN)]
    v1 = hx_s[:, 1:2] ^ hw_s[1:2, pl.ds(j * _TN, _TN)]
    m0 = (v0 - _C_ONES) & ~v0 & _C_HIGH
    m1 = (v1 - _C_ONES) & ~v1 & _C_HIGH
    mask = (m0 | m1) != 0
    out_ref[...] = jnp.where(mask, dense + b_ref[...], 0.0)


def kernel(x, W, b, proj):
    B, S, D = x.shape
    BS = B * S
    xf = x.reshape(BS, D)
    projM = proj.reshape(_T * _H, D)
    projT = projM.T
    b2 = b.reshape(1, _N)
    out = pl.pallas_call(
        _body,
        grid=(BS // _TS, _N // _TN),
        in_specs=[
            pl.BlockSpec((_TS, D), lambda i, j: (i, 0)),
            pl.BlockSpec((_TN, D), lambda i, j: (j, 0)),
            pl.BlockSpec((1, _TN), lambda i, j: (0, j)),
            pl.BlockSpec((D, _T * _H), lambda i, j: (0, 0)),
            pl.BlockSpec((_T * _H, D), lambda i, j: (0, 0)),
            pl.BlockSpec((_T * _H, 128), lambda i, j: (0, 0)),
            pl.BlockSpec((_T, _T * _H), lambda i, j: (0, 0)),
        ],
        out_specs=pl.BlockSpec((_TS, _TN), lambda i, j: (i, j)),
        out_shape=jax.ShapeDtypeStruct((BS, _N), jnp.float32),
        scratch_shapes=[
            pltpu.VMEM((_TS, 128), jnp.int32),
            pltpu.VMEM((_T, _N), jnp.int32),
        ],
    )(xf, W, b2, jnp.asarray(projT), projM, jnp.asarray(_PMAT),
      jnp.asarray(_PMAT_T8))
    return out.reshape(B, S, _N)
